# baseline (device time: 224897 ns/iter reference)
import functools

import jax
import jax.numpy as jnp
from jax import lax
from jax.experimental import pallas as pl
from jax.experimental.pallas import tpu as pltpu

N_DEV = 8
N_STEPS = N_DEV - 1
N_SLOTS = 4

B = (
    (2, 1, 4, 1, 2, 1, 4),
    (4, 2, 1, 2, 4, 2, 1),
    (1, 4, 2, 4, 1, 4, 2),
)
PFX = tuple(
    tuple(functools.reduce(lambda a, b: a ^ b, bits[:k])
          for k in range(1, N_STEPS + 1))
    for bits in B
)

PATHS = (
    (0, 0, 64),
    (1, 176, 56),
    (2, 344, 56),
    (0, 64, 56),
    (1, 232, 56),
    (2, 400, 56),
    (0, 120, 56),
    (1, 288, 56),
    (2, 456, 56),
)
N_PATHS = len(PATHS)


def _pos2v(p):
    return (p & 4) | ((p & 3) ^ ((p >> 1) & 1))


def _v2pos(v):
    return (v & 4) | (v & 2) | ((v ^ (v >> 1)) & 1)


def kernel(x, w_mat):
    m_per, k_dim = x.shape
    _, n_per = w_mat.shape

    def body(x_ref, w_ref, out_ref, *scratch):
        bufs = scratch[0::4]
        send_sems = scratch[1::4]
        recv_sems = scratch[2::4]
        cred_sems = scratch[3::4]

        my = lax.axis_index("i")
        myv = _pos2v(my)
        axis_partners = [_v2pos(myv ^ bit) for bit in (1, 2, 4)]

        barrier = pltpu.get_barrier_semaphore()
        for p in axis_partners:
            pl.semaphore_signal(barrier, inc=1, device_id=(p,),
                                device_id_type=pl.DeviceIdType.MESH)
        pl.semaphore_wait(barrier, 3)

        def partner(i, k):
            return _v2pos(myv ^ B[PATHS[i][0]][k - 1])

        def desc(i, k):
            _, row0, nrows = PATHS[i]
            slot = (k - 1) % N_SLOTS
            if k == 1:
                src = x_ref.at[pl.ds(row0, nrows), :]
            else:
                src = bufs[i].at[(k - 2) % N_SLOTS]
            return pltpu.make_async_remote_copy(
                src_ref=src,
                dst_ref=bufs[i].at[slot],
                send_sem=send_sems[i].at[slot],
                recv_sem=recv_sems[i].at[slot],
                device_id=(partner(i, k),),
                device_id_type=pl.DeviceIdType.MESH,
            )

        for i in range(N_PATHS):
            desc(i, 1).start()

        acc = jnp.dot(x_ref[...], w_ref[...],
                      preferred_element_type=jnp.float32)
        out_ref[pl.ds(my * m_per, m_per), :] = jnp.maximum(acc, 0.0)

        def gemm_store(i, k):
            cls, row0, nrows = PATHS[i]
            o = _v2pos(myv ^ PFX[cls][k - 1])
            y = jnp.dot(bufs[i][(k - 1) % N_SLOTS], w_ref[...],
                        preferred_element_type=jnp.float32)
            out_ref[pl.ds(o * m_per + row0, nrows), :] = jnp.maximum(y, 0.0)

        for k in range(1, N_STEPS + 1):
            for i in range(N_PATHS):
                desc(i, k).wait_recv()
                if k + 1 <= N_STEPS:
                    if k + 1 > N_SLOTS:
                        pl.semaphore_wait(cred_sems[i].at[k % N_SLOTS], 1)
                    desc(i, k + 1).start()
                else:
                    gemm_store(i, k)

            for i in range(N_PATHS):
                desc(i, k).wait_send()
                c = k - 1
                if 1 <= c <= N_STEPS - N_SLOTS:
                    pl.semaphore_signal(
                        cred_sems[i].at[(c + 3) % N_SLOTS], inc=1,
                        device_id=(partner(i, c + 4),),
                        device_id_type=pl.DeviceIdType.MESH)

            if k < N_STEPS:
                for i in range(N_PATHS):
                    gemm_store(i, k)

    scratch_shapes = []
    for _, _, nrows in PATHS:
        scratch_shapes += [
            pltpu.VMEM((N_SLOTS, nrows, x.shape[1]), jnp.float32),
            pltpu.SemaphoreType.DMA((N_SLOTS,)),
            pltpu.SemaphoreType.DMA((N_SLOTS,)),
            pltpu.SemaphoreType.REGULAR((N_SLOTS,)),
        ]

    out_shape = jax.ShapeDtypeStruct((N_DEV * m_per, n_per), jnp.float32)
    return pl.pallas_call(
        body,
        out_shape=out_shape,
        in_specs=[
            pl.BlockSpec(memory_space=pltpu.VMEM),
            pl.BlockSpec(memory_space=pltpu.VMEM),
        ],
        out_specs=pl.BlockSpec(memory_space=pltpu.VMEM),
        scratch_shapes=scratch_shapes,
        compiler_params=pltpu.CompilerParams(
            collective_id=0,
            vmem_limit_bytes=46 * 1024 * 1024,
        ),
    )(x, w_mat)


# device time: 224435 ns/iter; 1.0021x vs baseline; 1.0021x over previous
import functools

import jax
import jax.numpy as jnp
from jax import lax
from jax.experimental import pallas as pl
from jax.experimental.pallas import tpu as pltpu

N_DEV = 8
N_STEPS = N_DEV - 1
N_SLOTS = 4

B = (
    (2, 1, 4, 1, 2, 1, 4),
    (4, 2, 1, 2, 4, 2, 1),
    (1, 4, 2, 4, 1, 4, 2),
)
PFX = tuple(
    tuple(functools.reduce(lambda a, b: a ^ b, bits[:k])
          for k in range(1, N_STEPS + 1))
    for bits in B
)

PATHS = (
    (0, 0, 88),
    (1, 176, 88),
    (2, 344, 88),
    (0, 88, 88),
    (1, 264, 80),
    (2, 432, 80),
)
N_PATHS = len(PATHS)


def _pos2v(p):
    return (p & 4) | ((p & 3) ^ ((p >> 1) & 1))


def _v2pos(v):
    return (v & 4) | (v & 2) | ((v ^ (v >> 1)) & 1)


def kernel(x, w_mat):
    m_per, k_dim = x.shape
    _, n_per = w_mat.shape

    def body(x_ref, w_ref, out_ref, *scratch):
        bufs = scratch[0::4]
        send_sems = scratch[1::4]
        recv_sems = scratch[2::4]
        cred_sems = scratch[3::4]

        my = lax.axis_index("i")
        myv = _pos2v(my)
        axis_partners = [_v2pos(myv ^ bit) for bit in (1, 2, 4)]

        barrier = pltpu.get_barrier_semaphore()
        for p in axis_partners:
            pl.semaphore_signal(barrier, inc=1, device_id=(p,),
                                device_id_type=pl.DeviceIdType.MESH)
        pl.semaphore_wait(barrier, 3)

        def partner(i, k):
            return _v2pos(myv ^ B[PATHS[i][0]][k - 1])

        def desc(i, k):
            _, row0, nrows = PATHS[i]
            slot = (k - 1) % N_SLOTS
            if k == 1:
                src = x_ref.at[pl.ds(row0, nrows), :]
            else:
                src = bufs[i].at[(k - 2) % N_SLOTS]
            return pltpu.make_async_remote_copy(
                src_ref=src,
                dst_ref=bufs[i].at[slot],
                send_sem=send_sems[i].at[slot],
                recv_sem=recv_sems[i].at[slot],
                device_id=(partner(i, k),),
                device_id_type=pl.DeviceIdType.MESH,
            )

        for i in range(N_PATHS):
            desc(i, 1).start()

        acc = jnp.dot(x_ref[...], w_ref[...],
                      preferred_element_type=jnp.float32)
        out_ref[pl.ds(my * m_per, m_per), :] = jnp.maximum(acc, 0.0)

        def gemm_store(i, k):
            cls, row0, nrows = PATHS[i]
            o = _v2pos(myv ^ PFX[cls][k - 1])
            y = jnp.dot(bufs[i][(k - 1) % N_SLOTS], w_ref[...],
                        preferred_element_type=jnp.float32)
            out_ref[pl.ds(o * m_per + row0, nrows), :] = jnp.maximum(y, 0.0)

        for k in range(1, N_STEPS + 1):
            for i in range(N_PATHS):
                desc(i, k).wait_recv()
                if k + 1 <= N_STEPS:
                    if k + 1 > N_SLOTS:
                        pl.semaphore_wait(cred_sems[i].at[k % N_SLOTS], 1)
                    desc(i, k + 1).start()
                else:
                    gemm_store(i, k)

            for i in range(N_PATHS):
                desc(i, k).wait_send()
                c = k - 1
                if 1 <= c <= N_STEPS - N_SLOTS:
                    pl.semaphore_signal(
                        cred_sems[i].at[(c + 3) % N_SLOTS], inc=1,
                        device_id=(partner(i, c + 4),),
                        device_id_type=pl.DeviceIdType.MESH)

            if k < N_STEPS:
                for i in range(N_PATHS):
                    gemm_store(i, k)

    scratch_shapes = []
    for _, _, nrows in PATHS:
        scratch_shapes += [
            pltpu.VMEM((N_SLOTS, nrows, x.shape[1]), jnp.float32),
            pltpu.SemaphoreType.DMA((N_SLOTS,)),
            pltpu.SemaphoreType.DMA((N_SLOTS,)),
            pltpu.SemaphoreType.REGULAR((N_SLOTS,)),
        ]

    out_shape = jax.ShapeDtypeStruct((N_DEV * m_per, n_per), jnp.float32)
    return pl.pallas_call(
        body,
        out_shape=out_shape,
        in_specs=[
            pl.BlockSpec(memory_space=pltpu.VMEM),
            pl.BlockSpec(memory_space=pltpu.VMEM),
        ],
        out_specs=pl.BlockSpec(memory_space=pltpu.VMEM),
        scratch_shapes=scratch_shapes,
        compiler_params=pltpu.CompilerParams(
            collective_id=0,
            vmem_limit_bytes=46 * 1024 * 1024,
        ),
    )(x, w_mat)
